# baseline (speedup 1.0000x reference)
import functools

import jax
import jax.numpy as jnp
from jax import lax
from jax.experimental import pallas as pl
from jax.experimental.pallas import tpu as pltpu

_HIDDEN = 1000
_BATCH = 4096
_SEQ = 26
_BB = 128


def _onehot_block(x_ref, o_ref):
    idx = x_ref[...]
    classes = lax.broadcasted_iota(jnp.int32, (_BB, _SEQ, _HIDDEN), 2)
    o_ref[...] = (classes == idx[:, :, None]).astype(jnp.float32)


@jax.jit
def kernel(x):
    return pl.pallas_call(
        _onehot_block,
        grid=(_BATCH // _BB,),
        in_specs=[pl.BlockSpec((_BB, _SEQ), lambda i: (i, 0))],
        out_specs=pl.BlockSpec((_BB, _SEQ, _HIDDEN), lambda i: (i, 0, 0)),
        out_shape=jax.ShapeDtypeStruct((_BATCH, _SEQ, _HIDDEN), jnp.float32),
        compiler_params=pltpu.CompilerParams(
            dimension_semantics=("parallel",),
        ),
    )(x.astype(jnp.int32))
